# baseline (speedup 1.0000x reference)
import jax
import jax.numpy as jnp
from jax.experimental import pallas as pl
from jax.experimental.pallas import tpu as pltpu

NUM_TIMESTEPS = 1000
BETA_START = 1e-4
BETA_END = 0.02


def _schedule_tables():
    betas = jnp.linspace(BETA_START, BETA_END, NUM_TIMESTEPS, dtype=jnp.float32)
    alphas = 1.0 - betas
    alphas_cumprod = jnp.cumprod(alphas)
    sqrt_ac = jnp.sqrt(alphas_cumprod)
    sqrt_1mac = jnp.sqrt(1.0 - alphas_cumprod)
    return sqrt_ac, sqrt_1mac


def _body(ts_ref, ac_ref, mac_ref, x_ref, n_ref, o_ref):
    b = pl.program_id(0)
    t = ts_ref[b]
    a = ac_ref[t]
    c = mac_ref[t]
    o_ref[...] = a * x_ref[...] + c * n_ref[...]


def kernel(original_samples, noise, timesteps):
    B = original_samples.shape[0]
    F = original_samples.shape[1] * original_samples.shape[2] * original_samples.shape[3]
    R = F // 128
    x = original_samples.reshape(B, R, 128)
    n = noise.reshape(B, R, 128)
    sqrt_ac, sqrt_1mac = _schedule_tables()
    ts = timesteps.astype(jnp.int32)

    grid_spec = pltpu.PrefetchScalarGridSpec(
        num_scalar_prefetch=3,
        grid=(B,),
        in_specs=[
            pl.BlockSpec((1, R, 128), lambda b, *_: (b, 0, 0)),
            pl.BlockSpec((1, R, 128), lambda b, *_: (b, 0, 0)),
        ],
        out_specs=pl.BlockSpec((1, R, 128), lambda b, *_: (b, 0, 0)),
    )
    out = pl.pallas_call(
        _body,
        grid_spec=grid_spec,
        out_shape=jax.ShapeDtypeStruct((B, R, 128), jnp.float32),
        compiler_params=pltpu.CompilerParams(
            dimension_semantics=("parallel",),
        ),
    )(ts, sqrt_ac, sqrt_1mac, x, n)
    return out.reshape(original_samples.shape)


# speedup vs baseline: 1.1632x; 1.1632x over previous
import jax
import jax.numpy as jnp
from jax.experimental import pallas as pl
from jax.experimental.pallas import tpu as pltpu

NUM_TIMESTEPS = 1000
BETA_START = 1e-4
BETA_END = 0.02

SAMPLES_PER_BLOCK = 8


def _schedule_tables():
    betas = jnp.linspace(BETA_START, BETA_END, NUM_TIMESTEPS, dtype=jnp.float32)
    alphas = 1.0 - betas
    alphas_cumprod = jnp.cumprod(alphas)
    sqrt_ac = jnp.sqrt(alphas_cumprod)
    sqrt_1mac = jnp.sqrt(1.0 - alphas_cumprod)
    return sqrt_ac, sqrt_1mac


def _body(ts_ref, ac_ref, mac_ref, x_ref, n_ref, o_ref):
    g = pl.program_id(0)
    base = g * SAMPLES_PER_BLOCK
    a_s = []
    c_s = []
    for i in range(SAMPLES_PER_BLOCK):
        t = ts_ref[base + i]
        a_s.append(ac_ref[t])
        c_s.append(mac_ref[t])
    a_vec = jnp.stack(a_s).reshape(SAMPLES_PER_BLOCK, 1, 1)
    c_vec = jnp.stack(c_s).reshape(SAMPLES_PER_BLOCK, 1, 1)
    o_ref[...] = a_vec * x_ref[...] + c_vec * n_ref[...]


def kernel(original_samples, noise, timesteps):
    B = original_samples.shape[0]
    F = original_samples.shape[1] * original_samples.shape[2] * original_samples.shape[3]
    R = F // 128
    S = SAMPLES_PER_BLOCK
    x = original_samples.reshape(B, R, 128)
    n = noise.reshape(B, R, 128)
    sqrt_ac, sqrt_1mac = _schedule_tables()
    ts = timesteps.astype(jnp.int32)

    grid_spec = pltpu.PrefetchScalarGridSpec(
        num_scalar_prefetch=3,
        grid=(B // S,),
        in_specs=[
            pl.BlockSpec((S, R, 128), lambda b, *_: (b, 0, 0)),
            pl.BlockSpec((S, R, 128), lambda b, *_: (b, 0, 0)),
        ],
        out_specs=pl.BlockSpec((S, R, 128), lambda b, *_: (b, 0, 0)),
    )
    out = pl.pallas_call(
        _body,
        grid_spec=grid_spec,
        out_shape=jax.ShapeDtypeStruct((B, R, 128), jnp.float32),
        compiler_params=pltpu.CompilerParams(
            dimension_semantics=("parallel",),
        ),
    )(ts, sqrt_ac, sqrt_1mac, x, n)
    return out.reshape(original_samples.shape)
